# Initial kernel scaffold; baseline (speedup 1.0000x reference)
import jax
import jax.numpy as jnp
from jax.experimental import pallas as pl


def kernel(input, We0, be0, We1, be1, We2, be2, Wd0, bd0, Wd1, bd1, Wd2, bd2, codebook):
    raise NotImplementedError("write your pallas kernel here")


# speedup vs baseline: 1.1409x; 1.1409x over previous
import jax
import jax.numpy as jnp
from jax.experimental import pallas as pl
from jax.experimental.pallas import tpu as pltpu

_B = 16384
_COND = 256
_ACT = 32
_DIN = _COND + _ACT
_H0, _H1 = 512, 256
_EMB = 64
_K = 1024
_BLK = 1024


def _elu(x):
    return jnp.where(x > 0, x, jnp.exp(x) - 1.0)


def _fused(x_ref, We0, be0, We1, be1, We2, be2, Wd0, bd0, Wd1, bd1, Wd2, bd2,
           cb_ref, recon_ref, idx_ref, q_ref, e_ref, rec_ref, perp_ref,
           counts, sqacc, recacc):
    i = pl.program_id(0)
    n = pl.num_programs(0)

    @pl.when(i == 0)
    def _init():
        counts[...] = jnp.zeros_like(counts)
        sqacc[0, 0] = 0.0
        recacc[0, 0] = 0.0

    x = x_ref[...]
    h = _elu(jnp.dot(x, We0[...], preferred_element_type=jnp.float32) + be0[...])
    h = _elu(jnp.dot(h, We1[...], preferred_element_type=jnp.float32) + be1[...])
    z = jnp.dot(h, We2[...], preferred_element_type=jnp.float32) + be2[...]

    zn = z / (jnp.sqrt(jnp.sum(z * z, axis=-1, keepdims=True)) + 1e-12)
    cb = cb_ref[...]
    cbn = cb / (jnp.sqrt(jnp.sum(cb * cb, axis=-1, keepdims=True)) + 1e-12)
    dist = jax.lax.dot_general(zn, cbn, (((1,), (1,)), ((), ())),
                               preferred_element_type=jnp.float32)
    idx = jnp.argmax(dist, axis=-1).astype(jnp.int32)
    onehot = (jax.lax.broadcasted_iota(jnp.int32, (_BLK, _K), 1)
              == idx[:, None]).astype(jnp.float32)
    quant = jnp.dot(onehot, cb, preferred_element_type=jnp.float32)

    counts[...] += jnp.sum(onehot, axis=0, keepdims=True)
    diff = quant - z
    sqacc[0, 0] += jnp.sum(diff * diff)

    cond = x[:, :_COND]
    act = x[:, _COND:]
    h = _elu(jnp.dot(cond, Wd0[0:_COND, :], preferred_element_type=jnp.float32)
             + jnp.dot(quant, Wd0[_COND:_COND + _EMB, :], preferred_element_type=jnp.float32)
             + bd0[...])
    h = _elu(jnp.dot(h, Wd1[...], preferred_element_type=jnp.float32) + bd1[...])
    recon = jnp.dot(h, Wd2[...], preferred_element_type=jnp.float32) + bd2[...]
    rerr = recon - act
    recacc[0, 0] += jnp.sum(rerr * rerr)

    recon_ref[...] = recon
    idx_ref[...] = idx

    @pl.when(i == n - 1)
    def _fini():
        q = sqacc[0, 0] / (_B * _EMB)
        q_ref[...] = jnp.full((1, 1), q, dtype=jnp.float32)
        e_ref[...] = jnp.full((1, 1), 0.25 * q, dtype=jnp.float32)
        rec_ref[...] = jnp.full((1, 1), recacc[0, 0] / (_B * _ACT), dtype=jnp.float32)
        p = counts[...] / _B
        perp = jnp.exp(-jnp.sum(p * jnp.log(p + 1e-10)))
        perp_ref[...] = jnp.full((1, 1), perp, dtype=jnp.float32)


def kernel(input, We0, be0, We1, be1, We2, be2, Wd0, bd0, Wd1, bd1, Wd2, bd2, codebook):
    nblk = _B // _BLK
    rep = lambda *shape: pl.BlockSpec(shape, lambda i: tuple(0 for _ in shape))
    out = pl.pallas_call(
        _fused,
        grid=(nblk,),
        in_specs=[
            pl.BlockSpec((_BLK, _DIN), lambda i: (i, 0)),
            rep(_DIN, _H0), rep(_H0),
            rep(_H0, _H1), rep(_H1),
            rep(_H1, _EMB), rep(_EMB),
            rep(_COND + _EMB, _H1), rep(_H1),
            rep(_H1, _H0), rep(_H0),
            rep(_H0, _ACT), rep(_ACT),
            rep(_K, _EMB),
        ],
        out_specs=[
            pl.BlockSpec((_BLK, _ACT), lambda i: (i, 0)),
            pl.BlockSpec((_BLK,), lambda i: (i,)),
            pl.BlockSpec((1, 1), lambda i: (0, 0)),
            pl.BlockSpec((1, 1), lambda i: (0, 0)),
            pl.BlockSpec((1, 1), lambda i: (0, 0)),
            pl.BlockSpec((1, 1), lambda i: (0, 0)),
        ],
        out_shape=[
            jax.ShapeDtypeStruct((_B, _ACT), jnp.float32),
            jax.ShapeDtypeStruct((_B,), jnp.int32),
            jax.ShapeDtypeStruct((1, 1), jnp.float32),
            jax.ShapeDtypeStruct((1, 1), jnp.float32),
            jax.ShapeDtypeStruct((1, 1), jnp.float32),
            jax.ShapeDtypeStruct((1, 1), jnp.float32),
        ],
        scratch_shapes=[
            pltpu.VMEM((1, _K), jnp.float32),
            pltpu.SMEM((1, 1), jnp.float32),
            pltpu.SMEM((1, 1), jnp.float32),
        ],
        compiler_params=pltpu.CompilerParams(
            dimension_semantics=("arbitrary",),
        ),
    )(input, We0, be0, We1, be1, We2, be2, Wd0, bd0, Wd1, bd1, Wd2, bd2, codebook)
    recon, idx, q, e, rec, perp = out
    return (recon, idx, q[0, 0], e[0, 0], rec[0, 0], perp[0, 0])
